# baseline (speedup 1.0000x reference)
import functools

import jax
import jax.numpy as jnp
from jax import lax
from jax.experimental import pallas as pl
from jax.experimental.pallas import tpu as pltpu
from jax.experimental.pallas import tpu_sc as plsc

GROUP_NUM = 1000000
EMBED = 64
LATENT = 128
BATCH = 16384

_NC = 2
_NS = 16
_NW = _NC * _NS
_B_PER_W = BATCH // _NW
_UNROLL = 16


def _sc_gather_pairs(table2, idx2):
    mesh = plsc.VectorSubcoreMesh(core_axis_name="c", subcore_axis_name="s")

    @functools.partial(
        pl.kernel,
        mesh=mesh,
        out_type=jax.ShapeDtypeStruct((BATCH, 2 * EMBED), jnp.float32),
        scratch_types=[
            pltpu.VMEM((_B_PER_W,), jnp.int32),
            pltpu.VMEM((_B_PER_W, 2 * EMBED), jnp.float32),
            pltpu.SemaphoreType.DMA,
        ],
    )
    def k(table_hbm, idx_hbm, out_hbm, idx_v, rows_v, sem):
        wid = lax.axis_index("s") * _NC + lax.axis_index("c")
        base = wid * _B_PER_W
        pltpu.sync_copy(idx_hbm.at[wid], idx_v)

        def body(j, carry):
            vec = idx_v[pl.ds(j * _UNROLL, _UNROLL)]
            for u in range(_UNROLL):
                q = vec[u]
                pltpu.async_copy(
                    table_hbm.at[pl.ds(q, 1)],
                    rows_v.at[pl.ds(j * _UNROLL + u, 1)],
                    sem,
                )
            return carry

        lax.fori_loop(0, _B_PER_W // _UNROLL, body, 0)
        pltpu.make_async_copy(
            table_hbm.at[pl.ds(0, _B_PER_W)], rows_v, sem
        ).wait()
        pltpu.sync_copy(rows_v, out_hbm.at[pl.ds(base, _B_PER_W)])

    return k(table2, idx2)


def _fuse_body(pr_ref, par_ref, tt_ref, vt_ref, w1_ref, w2_ref, w3_ref,
               b_ref, o_ref):
    pairs = pr_ref[...]
    p = par_ref[...]
    g = pairs[:, :EMBED] + p * (pairs[:, EMBED:] - pairs[:, :EMBED])
    dn = (((0,), (0,)), ((), ()))
    acc = jnp.dot(g, w1_ref[...], preferred_element_type=jnp.float32)
    acc += lax.dot_general(tt_ref[...], w2_ref[...], dn,
                           preferred_element_type=jnp.float32)
    acc += lax.dot_general(vt_ref[...], w3_ref[...], dn,
                           preferred_element_type=jnp.float32)
    o_ref[...] = acc + b_ref[...]


def _tc_fuse(pairs, parity, tt, vt, w1, w2, w3, b2):
    bm = 2048
    grid = (BATCH // bm,)
    return pl.pallas_call(
        _fuse_body,
        grid=grid,
        in_specs=[
            pl.BlockSpec((bm, 2 * EMBED), lambda i: (i, 0)),
            pl.BlockSpec((bm, 1), lambda i: (i, 0)),
            pl.BlockSpec((EMBED, bm), lambda i: (0, i)),
            pl.BlockSpec((EMBED, bm), lambda i: (0, i)),
            pl.BlockSpec((EMBED, LATENT), lambda i: (0, 0)),
            pl.BlockSpec((EMBED, LATENT), lambda i: (0, 0)),
            pl.BlockSpec((EMBED, LATENT), lambda i: (0, 0)),
            pl.BlockSpec((1, LATENT), lambda i: (0, 0)),
        ],
        out_specs=pl.BlockSpec((bm, LATENT), lambda i: (i, 0)),
        out_shape=jax.ShapeDtypeStruct((BATCH, LATENT), jnp.float32),
    )(pairs, parity, tt, vt, w1, w2, w3, b2)


@jax.jit
def kernel(group_indices, txt_embed, vision_embed, table, W, b):
    idx = group_indices.astype(jnp.int32)
    q2 = lax.shift_right_logical(idx, 1).reshape(_NW, _B_PER_W)
    parity = (idx & 1).astype(jnp.float32).reshape(BATCH, 1)
    table2 = table.reshape(GROUP_NUM // 2, 2 * EMBED)
    pairs = _sc_gather_pairs(table2, q2)
    w1 = W[:EMBED]
    w2 = W[EMBED:2 * EMBED]
    w3 = W[2 * EMBED:]
    return _tc_fuse(pairs, parity, txt_embed.T, vision_embed.T,
                    w1, w2, w3, b.reshape(1, LATENT))


# speedup vs baseline: 1.7478x; 1.7478x over previous
import functools

import jax
import jax.numpy as jnp
from jax import lax
from jax.experimental import pallas as pl
from jax.experimental.pallas import tpu as pltpu
from jax.experimental.pallas import tpu_sc as plsc

GROUP_NUM = 1000000
EMBED = 64
LATENT = 128
BATCH = 16384

_TBM = 2048
_HSPLIT = 244 * _TBM
_HROWS = 245 * _TBM
_NC = 2
_NS = 16
_NW = _NC * _NS
_B_PER_W = BATCH // _NW
_UNROLL = 16


def _t_body(xa_ref, xb_ref, o_ref):
    o_ref[...] = jnp.concatenate(
        [xa_ref[...].T, xb_ref[...].T], axis=1)


def _tc_transpose(xt):
    grid = (_HROWS // _TBM,)
    nb = _HSPLIT // _TBM
    return pl.pallas_call(
        _t_body,
        grid=grid,
        in_specs=[
            pl.BlockSpec((EMBED, _TBM), lambda i: (0, i)),
            pl.BlockSpec((EMBED, _TBM), lambda i, nb=nb: (0, i + nb)),
        ],
        out_specs=pl.BlockSpec((_TBM, 2 * EMBED), lambda i: (i, 0)),
        out_shape=jax.ShapeDtypeStruct((_HROWS, 2 * EMBED), jnp.float32),
    )(xt, xt)


def _sc_gather_pairs(table2, idx2):
    mesh = plsc.VectorSubcoreMesh(core_axis_name="c", subcore_axis_name="s")

    @functools.partial(
        pl.kernel,
        mesh=mesh,
        out_type=jax.ShapeDtypeStruct((BATCH, 2 * EMBED), jnp.float32),
        scratch_types=[
            pltpu.VMEM((_B_PER_W,), jnp.int32),
            pltpu.VMEM((_B_PER_W, 2 * EMBED), jnp.float32),
            pltpu.SemaphoreType.DMA,
        ],
    )
    def k(table_hbm, idx_hbm, out_hbm, idx_v, rows_v, sem):
        wid = lax.axis_index("s") * _NC + lax.axis_index("c")
        base = wid * _B_PER_W
        pltpu.sync_copy(idx_hbm.at[wid], idx_v)

        def body(j, carry):
            vec = idx_v[pl.ds(j * _UNROLL, _UNROLL)]
            for u in range(_UNROLL):
                q = vec[u]
                pltpu.async_copy(
                    table_hbm.at[pl.ds(q, 1)],
                    rows_v.at[pl.ds(j * _UNROLL + u, 1)],
                    sem,
                )
            return carry

        lax.fori_loop(0, _B_PER_W // _UNROLL, body, 0)
        pltpu.make_async_copy(
            table_hbm.at[pl.ds(0, _B_PER_W)], rows_v, sem
        ).wait()
        pltpu.sync_copy(rows_v, out_hbm.at[pl.ds(base, _B_PER_W)])

    return k(table2, idx2)


def _fuse_body(pr_ref, hf_ref, tt_ref, vt_ref, w1_ref, w2_ref, w3_ref,
               b_ref, o_ref):
    pairs = pr_ref[...]
    h = hf_ref[...]
    g = jnp.where(h > 0.5, pairs[:, EMBED:], pairs[:, :EMBED])
    dn = (((0,), (0,)), ((), ()))
    acc = jnp.dot(g, w1_ref[...], preferred_element_type=jnp.float32)
    acc += lax.dot_general(tt_ref[...], w2_ref[...], dn,
                           preferred_element_type=jnp.float32)
    acc += lax.dot_general(vt_ref[...], w3_ref[...], dn,
                           preferred_element_type=jnp.float32)
    o_ref[...] = acc + b_ref[...]


def _tc_fuse(pairs, half, tt, vt, w1, w2, w3, b2):
    bm = 2048
    grid = (BATCH // bm,)
    return pl.pallas_call(
        _fuse_body,
        grid=grid,
        in_specs=[
            pl.BlockSpec((bm, 2 * EMBED), lambda i: (i, 0)),
            pl.BlockSpec((bm, 1), lambda i: (i, 0)),
            pl.BlockSpec((EMBED, bm), lambda i: (0, i)),
            pl.BlockSpec((EMBED, bm), lambda i: (0, i)),
            pl.BlockSpec((EMBED, LATENT), lambda i: (0, 0)),
            pl.BlockSpec((EMBED, LATENT), lambda i: (0, 0)),
            pl.BlockSpec((EMBED, LATENT), lambda i: (0, 0)),
            pl.BlockSpec((1, LATENT), lambda i: (0, 0)),
        ],
        out_specs=pl.BlockSpec((bm, LATENT), lambda i: (i, 0)),
        out_shape=jax.ShapeDtypeStruct((BATCH, LATENT), jnp.float32),
    )(pairs, half, tt, vt, w1, w2, w3, b2)


@jax.jit
def kernel(group_indices, txt_embed, vision_embed, table, W, b):
    idx = group_indices.astype(jnp.int32)
    half = (idx >= _HSPLIT).astype(jnp.int32)
    q2 = (idx - half * _HSPLIT).reshape(_NW, _B_PER_W)
    halff = half.astype(jnp.float32).reshape(BATCH, 1)
    table2 = _tc_transpose(table.T)
    pairs = _sc_gather_pairs(table2, q2)
    w1 = W[:EMBED]
    w2 = W[EMBED:2 * EMBED]
    w3 = W[2 * EMBED:]
    return _tc_fuse(pairs, halff, txt_embed.T, vision_embed.T,
                    w1, w2, w3, b.reshape(1, LATENT))


# speedup vs baseline: 2.3959x; 1.3708x over previous
import functools

import jax
import jax.numpy as jnp
from jax import lax
from jax.experimental import pallas as pl
from jax.experimental.pallas import tpu as pltpu
from jax.experimental.pallas import tpu_sc as plsc

GROUP_NUM = 1000000
EMBED = 64
LATENT = 128
BATCH = 16384

_TBM = 8192
_HSPLIT = 61 * _TBM
_HROWS = 62 * _TBM
_NC = 2
_NS = 16
_NW = _NC * _NS
_B_PER_W = BATCH // _NW
_UNROLL = 16


def _t_body(xa_ref, xb_ref, o_ref):
    r = lax.broadcasted_iota(jnp.int32, (EMBED, EMBED), 0)
    c = lax.broadcasted_iota(jnp.int32, (EMBED, EMBED), 1)
    eye = (r == c).astype(jnp.float32)
    dn = (((0,), (0,)), ((), ()))
    o_ref[:, :EMBED] = lax.dot_general(xa_ref[...], eye, dn,
                                       preferred_element_type=jnp.float32)
    o_ref[:, EMBED:] = lax.dot_general(xb_ref[...], eye, dn,
                                       preferred_element_type=jnp.float32)


def _tc_transpose(xt):
    grid = (_HROWS // _TBM,)
    nb = _HSPLIT // _TBM
    return pl.pallas_call(
        _t_body,
        grid=grid,
        compiler_params=pltpu.CompilerParams(
            fuse_transposed_lhs_in_matmul=True),
        in_specs=[
            pl.BlockSpec((EMBED, _TBM), lambda i: (0, i)),
            pl.BlockSpec((EMBED, _TBM), lambda i, nb=nb: (0, i + nb)),
        ],
        out_specs=pl.BlockSpec((_TBM, 2 * EMBED), lambda i: (i, 0)),
        out_shape=jax.ShapeDtypeStruct((_HROWS, 2 * EMBED), jnp.float32),
    )(xt, xt)


def _sc_gather_pairs(table2, idx2):
    mesh = plsc.VectorSubcoreMesh(core_axis_name="c", subcore_axis_name="s")

    @functools.partial(
        pl.kernel,
        mesh=mesh,
        out_type=jax.ShapeDtypeStruct((BATCH, 2 * EMBED), jnp.float32),
        scratch_types=[
            pltpu.VMEM((_B_PER_W,), jnp.int32),
            pltpu.VMEM((_B_PER_W, 2 * EMBED), jnp.float32),
            pltpu.SemaphoreType.DMA,
        ],
    )
    def k(table_hbm, idx_hbm, out_hbm, idx_v, rows_v, sem):
        wid = lax.axis_index("s") * _NC + lax.axis_index("c")
        base = wid * _B_PER_W
        pltpu.sync_copy(idx_hbm.at[wid], idx_v)

        def body(j, carry):
            vec = idx_v[pl.ds(j * _UNROLL, _UNROLL)]
            for u in range(_UNROLL):
                q = vec[u]
                pltpu.async_copy(
                    table_hbm.at[pl.ds(q, 1)],
                    rows_v.at[pl.ds(j * _UNROLL + u, 1)],
                    sem,
                )
            return carry

        lax.fori_loop(0, _B_PER_W // _UNROLL, body, 0)
        pltpu.make_async_copy(
            table_hbm.at[pl.ds(0, _B_PER_W)], rows_v, sem
        ).wait()
        pltpu.sync_copy(rows_v, out_hbm.at[pl.ds(base, _B_PER_W)])

    return k(table2, idx2)


def _fuse_body(pr_ref, hf_ref, tt_ref, vt_ref, w1_ref, w2_ref, w3_ref,
               b_ref, o_ref):
    pairs = pr_ref[...]
    h = hf_ref[...]
    g = jnp.where(h > 0.5, pairs[:, EMBED:], pairs[:, :EMBED])
    dn = (((0,), (0,)), ((), ()))
    acc = jnp.dot(g, w1_ref[...], preferred_element_type=jnp.float32)
    acc += lax.dot_general(tt_ref[...], w2_ref[...], dn,
                           preferred_element_type=jnp.float32)
    acc += lax.dot_general(vt_ref[...], w3_ref[...], dn,
                           preferred_element_type=jnp.float32)
    o_ref[...] = acc + b_ref[...]


def _tc_fuse(pairs, half, tt, vt, w1, w2, w3, b2):
    bm = 2048
    grid = (BATCH // bm,)
    return pl.pallas_call(
        _fuse_body,
        grid=grid,
        in_specs=[
            pl.BlockSpec((bm, 2 * EMBED), lambda i: (i, 0)),
            pl.BlockSpec((bm, 1), lambda i: (i, 0)),
            pl.BlockSpec((EMBED, bm), lambda i: (0, i)),
            pl.BlockSpec((EMBED, bm), lambda i: (0, i)),
            pl.BlockSpec((EMBED, LATENT), lambda i: (0, 0)),
            pl.BlockSpec((EMBED, LATENT), lambda i: (0, 0)),
            pl.BlockSpec((EMBED, LATENT), lambda i: (0, 0)),
            pl.BlockSpec((1, LATENT), lambda i: (0, 0)),
        ],
        out_specs=pl.BlockSpec((bm, LATENT), lambda i: (i, 0)),
        out_shape=jax.ShapeDtypeStruct((BATCH, LATENT), jnp.float32),
    )(pairs, half, tt, vt, w1, w2, w3, b2)


@jax.jit
def kernel(group_indices, txt_embed, vision_embed, table, W, b):
    idx = group_indices.astype(jnp.int32)
    half = (idx >= _HSPLIT).astype(jnp.int32)
    q2 = (idx - half * _HSPLIT).reshape(_NW, _B_PER_W)
    halff = half.astype(jnp.float32).reshape(BATCH, 1)
    table2 = _tc_transpose(table.T)
    pairs = _sc_gather_pairs(table2, q2)
    w1 = W[:EMBED]
    w2 = W[EMBED:2 * EMBED]
    w3 = W[2 * EMBED:]
    return _tc_fuse(pairs, halff, txt_embed.T, vision_embed.T,
                    w1, w2, w3, b.reshape(1, LATENT))


# speedup vs baseline: 2.4011x; 1.0022x over previous
import functools

import jax
import jax.numpy as jnp
from jax import lax
from jax.experimental import pallas as pl
from jax.experimental.pallas import tpu as pltpu
from jax.experimental.pallas import tpu_sc as plsc

GROUP_NUM = 1000000
EMBED = 64
LATENT = 128
BATCH = 16384

_TBM = 8192
_HSPLIT = 61 * _TBM
_HROWS = 62 * _TBM
_NC = 2
_NS = 16
_NW = _NC * _NS
_B_PER_W = BATCH // _NW
_UNROLL = 16


def _t_body(xa_ref, xb_ref, o_ref):
    r = lax.broadcasted_iota(jnp.int32, (EMBED, EMBED), 0)
    c = lax.broadcasted_iota(jnp.int32, (EMBED, EMBED), 1)
    eye = (r == c).astype(jnp.float32)
    dn = (((0,), (0,)), ((), ()))
    o_ref[:, :EMBED] = lax.dot_general(xa_ref[...], eye, dn,
                                       preferred_element_type=jnp.float32)
    o_ref[:, EMBED:] = lax.dot_general(xb_ref[...], eye, dn,
                                       preferred_element_type=jnp.float32)


def _tc_transpose(xt):
    grid = (_HROWS // _TBM,)
    nb = _HSPLIT // _TBM
    return pl.pallas_call(
        _t_body,
        grid=grid,
        compiler_params=pltpu.CompilerParams(
            fuse_transposed_lhs_in_matmul=True),
        in_specs=[
            pl.BlockSpec((EMBED, _TBM), lambda i: (0, i)),
            pl.BlockSpec((EMBED, _TBM), lambda i, nb=nb: (0, i + nb)),
        ],
        out_specs=pl.BlockSpec((_TBM, 2 * EMBED), lambda i: (i, 0)),
        out_shape=jax.ShapeDtypeStruct((_HROWS, 2 * EMBED), jnp.float32),
    )(xt, xt)


def _sc_gather_pairs(table2, idx2):
    mesh = plsc.VectorSubcoreMesh(core_axis_name="c", subcore_axis_name="s")

    @functools.partial(
        pl.kernel,
        mesh=mesh,
        out_type=jax.ShapeDtypeStruct((BATCH, 2 * EMBED), jnp.float32),
        scratch_types=[
            pltpu.VMEM((_B_PER_W,), jnp.int32),
            pltpu.VMEM((_B_PER_W, 2 * EMBED), jnp.float32),
            pltpu.SemaphoreType.DMA,
        ],
    )
    def k(table_hbm, idx_hbm, out_hbm, idx_v, rows_v, sem):
        wid = lax.axis_index("s") * _NC + lax.axis_index("c")
        base = wid * _B_PER_W
        pltpu.sync_copy(idx_hbm.at[wid], idx_v)

        def body(j, carry):
            vec = idx_v[pl.ds(j * _UNROLL, _UNROLL)]
            for u in range(_UNROLL):
                q = vec[u]
                pltpu.async_copy(
                    table_hbm.at[pl.ds(q, 1)],
                    rows_v.at[pl.ds(j * _UNROLL + u, 1)],
                    sem,
                )
            return carry

        lax.fori_loop(0, _B_PER_W // _UNROLL, body, 0)
        pltpu.make_async_copy(
            table_hbm.at[pl.ds(0, _B_PER_W)], rows_v, sem
        ).wait()
        pltpu.sync_copy(rows_v, out_hbm.at[pl.ds(base, _B_PER_W)])

    return k(table2, idx2)


def _fuse_body(pr_ref, hf_ref, tt_ref, vt_ref, w1_ref, w2_ref, w3_ref,
               b_ref, o_ref):
    pairs = pr_ref[...]
    h = hf_ref[...]
    g = jnp.where(h > 0.5, pairs[:, EMBED:], pairs[:, :EMBED])
    dn = (((0,), (0,)), ((), ()))
    acc = jnp.dot(g, w1_ref[...], preferred_element_type=jnp.float32)
    acc += lax.dot_general(tt_ref[...], w2_ref[...], dn,
                           preferred_element_type=jnp.float32)
    acc += lax.dot_general(vt_ref[...], w3_ref[...], dn,
                           preferred_element_type=jnp.float32)
    o_ref[...] = acc + b_ref[...]


def _tc_fuse(pairs, half, tt, vt, w1, w2, w3, b2):
    bm = 4096
    grid = (BATCH // bm,)
    return pl.pallas_call(
        _fuse_body,
        grid=grid,
        in_specs=[
            pl.BlockSpec((bm, 2 * EMBED), lambda i: (i, 0)),
            pl.BlockSpec((bm, 1), lambda i: (i, 0)),
            pl.BlockSpec((EMBED, bm), lambda i: (0, i)),
            pl.BlockSpec((EMBED, bm), lambda i: (0, i)),
            pl.BlockSpec((EMBED, LATENT), lambda i: (0, 0)),
            pl.BlockSpec((EMBED, LATENT), lambda i: (0, 0)),
            pl.BlockSpec((EMBED, LATENT), lambda i: (0, 0)),
            pl.BlockSpec((1, LATENT), lambda i: (0, 0)),
        ],
        out_specs=pl.BlockSpec((bm, LATENT), lambda i: (i, 0)),
        out_shape=jax.ShapeDtypeStruct((BATCH, LATENT), jnp.float32),
    )(pairs, half, tt, vt, w1, w2, w3, b2)


@jax.jit
def kernel(group_indices, txt_embed, vision_embed, table, W, b):
    idx = group_indices.astype(jnp.int32)
    half = (idx >= _HSPLIT).astype(jnp.int32)
    q2 = (idx - half * _HSPLIT).reshape(_NW, _B_PER_W)
    halff = half.astype(jnp.float32).reshape(BATCH, 1)
    table2 = _tc_transpose(table.T)
    pairs = _sc_gather_pairs(table2, q2)
    w1 = W[:EMBED]
    w2 = W[EMBED:2 * EMBED]
    w3 = W[2 * EMBED:]
    return _tc_fuse(pairs, halff, txt_embed.T, vision_embed.T,
                    w1, w2, w3, b.reshape(1, LATENT))


# speedup vs baseline: 2.4521x; 1.0212x over previous
import functools

import jax
import jax.numpy as jnp
from jax import lax
from jax.experimental import pallas as pl
from jax.experimental.pallas import tpu as pltpu
from jax.experimental.pallas import tpu_sc as plsc

GROUP_NUM = 1000000
EMBED = 64
LATENT = 128
BATCH = 16384

_TBM = 8192
_HSPLIT = 61 * _TBM
_HROWS = 62 * _TBM
_NC = 2
_NS = 16
_NW = _NC * _NS
_B_PER_W = BATCH // _NW
_UNROLL = 16


def _t_body(xa_ref, xb_ref, o_ref):
    r = lax.broadcasted_iota(jnp.int32, (EMBED, EMBED), 0)
    c = lax.broadcasted_iota(jnp.int32, (EMBED, EMBED), 1)
    eye = (r == c).astype(jnp.float32)
    dn = (((0,), (0,)), ((), ()))
    o_ref[:, :EMBED] = lax.dot_general(xa_ref[...], eye, dn,
                                       preferred_element_type=jnp.float32)
    o_ref[:, EMBED:] = lax.dot_general(xb_ref[...], eye, dn,
                                       preferred_element_type=jnp.float32)


def _tc_transpose(xt):
    grid = (_HROWS // _TBM,)
    nb = _HSPLIT // _TBM
    return pl.pallas_call(
        _t_body,
        grid=grid,
        compiler_params=pltpu.CompilerParams(
            fuse_transposed_lhs_in_matmul=True),
        in_specs=[
            pl.BlockSpec((EMBED, _TBM), lambda i: (0, i)),
            pl.BlockSpec((EMBED, _TBM), lambda i, nb=nb: (0, i + nb)),
        ],
        out_specs=pl.BlockSpec((_TBM, 2 * EMBED), lambda i: (i, 0)),
        out_shape=jax.ShapeDtypeStruct((_HROWS, 2 * EMBED), jnp.float32),
    )(xt, xt)


def _sc_gather_pairs(table2, idx2):
    mesh = plsc.VectorSubcoreMesh(core_axis_name="c", subcore_axis_name="s")

    @functools.partial(
        pl.kernel,
        mesh=mesh,
        out_type=jax.ShapeDtypeStruct((BATCH, 2 * EMBED), jnp.float32),
        scratch_types=[
            pltpu.VMEM((_B_PER_W,), jnp.int32),
            pltpu.VMEM((_B_PER_W, 2 * EMBED), jnp.float32),
            pltpu.SemaphoreType.DMA,
        ],
    )
    def k(table_hbm, idx_hbm, out_hbm, idx_v, rows_v, sem):
        wid = lax.axis_index("s") * _NC + lax.axis_index("c")
        base = wid * _B_PER_W
        pltpu.sync_copy(idx_hbm.at[wid], idx_v)

        def body(j, carry):
            vec = idx_v[pl.ds(j * _UNROLL, _UNROLL)]
            for u in range(_UNROLL):
                q = vec[u]
                pltpu.async_copy(
                    table_hbm.at[pl.ds(q, 1)],
                    rows_v.at[pl.ds(j * _UNROLL + u, 1)],
                    sem,
                )
            return carry

        lax.fori_loop(0, _B_PER_W // _UNROLL, body, 0)
        pltpu.make_async_copy(
            table_hbm.at[pl.ds(0, _B_PER_W)], rows_v, sem
        ).wait()
        pltpu.sync_copy(rows_v, out_hbm.at[pl.ds(base, _B_PER_W)])

    return k(table2, idx2)


def _fuse_body(pr_ref, hf_ref, tt_ref, vt_ref, w1_ref, w2_ref, w3_ref,
               b_ref, o_ref):
    pairs = pr_ref[...]
    h = hf_ref[...].reshape(pairs.shape[0], 1)
    g = jnp.where(h > 0.5, pairs[:, EMBED:], pairs[:, :EMBED])
    dn = (((0,), (0,)), ((), ()))
    acc = jnp.dot(g, w1_ref[...], preferred_element_type=jnp.float32)
    acc += lax.dot_general(tt_ref[...], w2_ref[...], dn,
                           preferred_element_type=jnp.float32)
    acc += lax.dot_general(vt_ref[...], w3_ref[...], dn,
                           preferred_element_type=jnp.float32)
    o_ref[...] = acc + b_ref[...]


def _tc_fuse(pairs, half, tt, vt, w1, w2, w3, b2):
    bm = 4096
    grid = (BATCH // bm,)
    return pl.pallas_call(
        _fuse_body,
        grid=grid,
        in_specs=[
            pl.BlockSpec((bm, 2 * EMBED), lambda i: (i, 0)),
            pl.BlockSpec((1, bm), lambda i: (0, i)),
            pl.BlockSpec((EMBED, bm), lambda i: (0, i)),
            pl.BlockSpec((EMBED, bm), lambda i: (0, i)),
            pl.BlockSpec((EMBED, LATENT), lambda i: (0, 0)),
            pl.BlockSpec((EMBED, LATENT), lambda i: (0, 0)),
            pl.BlockSpec((EMBED, LATENT), lambda i: (0, 0)),
            pl.BlockSpec((1, LATENT), lambda i: (0, 0)),
        ],
        out_specs=pl.BlockSpec((bm, LATENT), lambda i: (i, 0)),
        out_shape=jax.ShapeDtypeStruct((BATCH, LATENT), jnp.float32),
    )(pairs, half, tt, vt, w1, w2, w3, b2)


@jax.jit
def kernel(group_indices, txt_embed, vision_embed, table, W, b):
    idx = group_indices.astype(jnp.int32)
    half = (idx >= _HSPLIT).astype(jnp.int32)
    q2 = (idx - half * _HSPLIT).reshape(_NW, _B_PER_W)
    halff = half.astype(jnp.float32).reshape(1, BATCH)
    table2 = _tc_transpose(table.T)
    pairs = _sc_gather_pairs(table2, q2)
    w1 = W[:EMBED]
    w2 = W[EMBED:2 * EMBED]
    w3 = W[2 * EMBED:]
    return _tc_fuse(pairs, halff, txt_embed.T, vision_embed.T,
                    w1, w2, w3, b.reshape(1, LATENT))
